# baseline (speedup 1.0000x reference)
import functools

import jax
import jax.numpy as jnp
from jax import lax
from jax.experimental import pallas as pl
from jax.experimental.pallas import tpu as pltpu
from jax.experimental.pallas import tpu_sc as plsc

NC = 2
NS = 16
NW = NC * NS
LANE = 16
EGROUP = 128
NUM_GROUPS_POOL = 16


def _round_up(v, m):
    return -(-v // m) * m


def _sc_mesh():
    return plsc.VectorSubcoreMesh(core_axis_name="c", subcore_axis_name="s")


def _make_deg_kernel(n_nodes, n_groups):
    gpw = n_groups // NW
    npad = _round_up(n_nodes + 1, NS * 8)
    stripe = npad // NS

    @functools.partial(
        pl.kernel,
        out_type=jax.ShapeDtypeStruct((NC, NS, stripe, LANE), jnp.float32),
        mesh=_sc_mesh(),
        scratch_types=[
            pltpu.VMEM((gpw, EGROUP), jnp.int32),
            pltpu.VMEM((EGROUP, LANE), jnp.float32),
            pltpu.VMEM((EGROUP, LANE), jnp.float32),
            pltpu.VMEM_SHARED((npad, LANE), jnp.float32),
        ],
        compiler_params=pltpu.CompilerParams(use_tc_tiling_on_sc=False),
    )
    def deg_kernel(dst_hbm, out_hbm, dstbuf, ones_v, zeros_v, acc_sh):
        cid = lax.axis_index("c")
        sid = lax.axis_index("s")
        wid = sid * NC + cid

        def fill(i, carry):
            ones_v[i, :] = jnp.full((LANE,), 1.0, jnp.float32)
            zeros_v[i, :] = jnp.zeros((LANE,), jnp.float32)
            return carry
        lax.fori_loop(0, EGROUP, fill, 0)

        base = sid * stripe
        nfull = stripe // EGROUP
        for k in range(nfull):
            pltpu.sync_copy(zeros_v, acc_sh.at[pl.ds(base + k * EGROUP, EGROUP)])
        rem = stripe - nfull * EGROUP
        if rem:
            pltpu.sync_copy(zeros_v.at[pl.ds(0, rem)],
                            acc_sh.at[pl.ds(base + nfull * EGROUP, rem)])
        plsc.subcore_barrier()

        pltpu.sync_copy(dst_hbm.at[pl.ds(wid * gpw, gpw)], dstbuf)

        def step(j, carry):
            pltpu.sync_copy(ones_v, acc_sh.at[dstbuf.at[j]], add=True)
            return carry
        lax.fori_loop(0, gpw, step, 0)
        plsc.subcore_barrier()

        pltpu.sync_copy(acc_sh.at[pl.ds(base, stripe)],
                        out_hbm.at[cid].at[sid])

    return deg_kernel


def _make_agg_kernel(n_nodes, n_groups, h_dim):
    gpw = n_groups // NW
    npad = _round_up(n_nodes + 1, NS * 8)
    stripe = npad // NS

    @functools.partial(
        pl.kernel,
        out_type=jax.ShapeDtypeStruct((NC, NS, stripe, h_dim), jnp.float32),
        mesh=_sc_mesh(),
        scratch_types=[
            pltpu.VMEM((gpw, EGROUP), jnp.int32),
            pltpu.VMEM((gpw, EGROUP), jnp.int32),
            pltpu.VMEM((2, EGROUP, h_dim), jnp.float32),
            pltpu.VMEM_SHARED((npad, h_dim), jnp.float32),
            pltpu.VMEM_SHARED((npad, h_dim), jnp.float32),
            pltpu.SemaphoreType.DMA,
            pltpu.SemaphoreType.DMA,
        ],
        compiler_params=pltpu.CompilerParams(use_tc_tiling_on_sc=False),
    )
    def agg_kernel(src_hbm, dst_hbm, h_hbm, out_hbm,
                   srcbuf, dstbuf, rows_v, acc_sh, h_sh,
                   sem_a, sem_b):
        cid = lax.axis_index("c")
        sid = lax.axis_index("s")
        wid = sid * NC + cid

        def zfill(i, carry):
            for k in range(h_dim // LANE):
                rows_v[0, i, pl.ds(k * LANE, LANE)] = jnp.zeros((LANE,), jnp.float32)
            return carry
        lax.fori_loop(0, EGROUP, zfill, 0)

        base = sid * stripe
        nfull = stripe // EGROUP
        for k in range(nfull):
            pltpu.sync_copy(rows_v.at[0],
                            acc_sh.at[pl.ds(base + k * EGROUP, EGROUP)])
        rem = stripe - nfull * EGROUP
        if rem:
            pltpu.sync_copy(rows_v.at[0].at[pl.ds(0, rem)],
                            acc_sh.at[pl.ds(base + nfull * EGROUP, rem)])

        nfull_h = n_nodes // EGROUP
        tail_h = n_nodes - nfull_h * EGROUP
        nck = -(-(nfull_h + (1 if tail_h else 0)) // NS)
        for kk in range(nck):
            k = kk * NS + sid

            @pl.when(k < nfull_h)
            def _():
                pltpu.sync_copy(h_hbm.at[pl.ds(k * EGROUP, EGROUP)],
                                h_sh.at[pl.ds(k * EGROUP, EGROUP)])
            if tail_h:
                @pl.when(k == nfull_h)
                def _():
                    pltpu.sync_copy(h_hbm.at[pl.ds(nfull_h * EGROUP, tail_h)],
                                    h_sh.at[pl.ds(nfull_h * EGROUP, tail_h)])
        plsc.subcore_barrier()

        pltpu.sync_copy(src_hbm.at[pl.ds(wid * gpw, gpw)], srcbuf)
        pltpu.sync_copy(dst_hbm.at[pl.ds(wid * gpw, gpw)], dstbuf)

        pltpu.async_copy(h_sh.at[srcbuf.at[0]], rows_v.at[0], sem_a)

        def step2(jj, carry):
            j = jj * 2
            use_hbm = (jj % 2) == 1

            @pl.when(use_hbm)
            def _():
                pltpu.async_copy(h_hbm.at[srcbuf.at[j + 1]], rows_v.at[1],
                                 sem_b)

            @pl.when(jnp.logical_not(use_hbm))
            def _():
                pltpu.async_copy(h_sh.at[srcbuf.at[j + 1]], rows_v.at[1],
                                 sem_b)
            pltpu.make_async_copy(h_sh.at[srcbuf.at[j]],
                                  rows_v.at[0], sem_a).wait()
            pltpu.sync_copy(rows_v.at[0], acc_sh.at[dstbuf.at[j]], add=True)

            @pl.when(jj + 1 < gpw // 2)
            def _():
                pltpu.async_copy(h_sh.at[srcbuf.at[j + 2]],
                                 rows_v.at[0], sem_a)
            pltpu.make_async_copy(h_sh.at[srcbuf.at[j + 1]],
                                  rows_v.at[1], sem_b).wait()
            pltpu.sync_copy(rows_v.at[1], acc_sh.at[dstbuf.at[j + 1]], add=True)
            return carry
        lax.fori_loop(0, gpw // 2, step2, 0)
        plsc.subcore_barrier()

        pltpu.sync_copy(acc_sh.at[pl.ds(base, stripe)],
                        out_hbm.at[cid].at[sid])

    return agg_kernel


def _dinv_from(deg_ref):
    deg = deg_ref[0, :, 0] + deg_ref[1, :, 0] + 1.0
    return lax.rsqrt(deg)


def _tc_matmul_scale(x, w, degp, bn):
    n, d = x.shape
    h = w.shape[1]

    def body(x_ref, w_ref, deg_ref, o_ref):
        dinv = _dinv_from(deg_ref)
        acc = jnp.dot(x_ref[...], w_ref[...], preferred_element_type=jnp.float32)
        o_ref[...] = acc * dinv[:, None]

    return pl.pallas_call(
        body,
        grid=(n // bn,),
        in_specs=[
            pl.BlockSpec((bn, d), lambda i: (i, 0)),
            pl.BlockSpec((d, h), lambda i: (0, 0)),
            pl.BlockSpec((2, bn, LANE), lambda i: (0, i, 0)),
        ],
        out_specs=pl.BlockSpec((bn, h), lambda i: (i, 0)),
        out_shape=jax.ShapeDtypeStruct((n, h), jnp.float32),
    )(x, w, degp)


def _tc_mid(accp, hp, degp, b, w2, bn):
    n, h = hp.shape
    h2 = w2.shape[1]

    def body(acc_ref, hp_ref, deg_ref, b_ref, w_ref, o_ref):
        dinv = _dinv_from(deg_ref)
        s = (acc_ref[0] + acc_ref[1] + hp_ref[...]) * dinv[:, None] + b_ref[...]
        hrelu = jnp.maximum(s, 0.0)
        o_ref[...] = jnp.dot(hrelu, w_ref[...],
                             preferred_element_type=jnp.float32) * dinv[:, None]

    return pl.pallas_call(
        body,
        grid=(n // bn,),
        in_specs=[
            pl.BlockSpec((2, bn, h), lambda i: (0, i, 0)),
            pl.BlockSpec((bn, h), lambda i: (i, 0)),
            pl.BlockSpec((2, bn, LANE), lambda i: (0, i, 0)),
            pl.BlockSpec((1, h), lambda i: (0, 0)),
            pl.BlockSpec((h, h2), lambda i: (0, 0)),
        ],
        out_specs=pl.BlockSpec((bn, h), lambda i: (i, 0)),
        out_shape=jax.ShapeDtypeStruct((n, h), jnp.float32),
    )(accp, hp, degp, b, w2)


def _tc_final(accp, hp, degp, b, batch3, wfc, bfc, bn):
    n, h = hp.shape
    c = wfc.shape[1]
    g = NUM_GROUPS_POOL
    nblk = n // bn

    def body(acc_ref, hp_ref, deg_ref, b_ref, batch_ref, wfc_ref, bfc_ref,
             o_ref, psum, cnt):
        i = pl.program_id(0)
        dinv = _dinv_from(deg_ref)
        s = (acc_ref[0] + acc_ref[1] + hp_ref[...]) * dinv[:, None] + b_ref[...]
        hrelu = jnp.maximum(s, 0.0)
        bvec = batch_ref[0, 0, :]
        oh = (bvec[:, None] == lax.broadcasted_iota(jnp.int32, (1, g), 1)
              ).astype(jnp.float32)
        contrib = lax.dot_general(oh, hrelu, (((0,), (0,)), ((), ())),
                                  preferred_element_type=jnp.float32)
        cnt_c = jnp.sum(oh, axis=0)[None, :]

        @pl.when(i == 0)
        def _():
            psum[...] = jnp.zeros_like(psum)
            cnt[...] = jnp.zeros_like(cnt)

        psum[...] += contrib
        cnt[...] += cnt_c

        @pl.when(i == nblk - 1)
        def _():
            pooled = psum[...] / jnp.maximum(cnt[...], 1.0)[0][:, None]
            o_ref[...] = jnp.dot(pooled, wfc_ref[...],
                                 preferred_element_type=jnp.float32) + bfc_ref[...]

    return pl.pallas_call(
        body,
        grid=(nblk,),
        in_specs=[
            pl.BlockSpec((2, bn, h), lambda i: (0, i, 0)),
            pl.BlockSpec((bn, h), lambda i: (i, 0)),
            pl.BlockSpec((2, bn, LANE), lambda i: (0, i, 0)),
            pl.BlockSpec((1, h), lambda i: (0, 0)),
            pl.BlockSpec((1, 1, bn), lambda i: (i, 0, 0)),
            pl.BlockSpec((h, c), lambda i: (0, 0)),
            pl.BlockSpec((1, c), lambda i: (0, 0)),
        ],
        out_specs=pl.BlockSpec((g, c), lambda i: (0, 0)),
        out_shape=jax.ShapeDtypeStruct((g, c), jnp.float32),
        scratch_shapes=[
            pltpu.VMEM((g, h), jnp.float32),
            pltpu.VMEM((1, g), jnp.float32),
        ],
    )(accp, hp, degp, b, batch3, wfc, bfc)


def kernel(x, edge_index, batch, W1, b1, W2, b2, Wfc, bfc):
    n, d = x.shape
    h_dim = W1.shape[1]
    e = edge_index.shape[1]
    bn = 2000

    ng = _round_up(_round_up(e, EGROUP) // EGROUP, NW * 8)
    epad = ng * EGROUP - e
    npad = _round_up(n + 1, NS * 8)
    src = jnp.concatenate(
        [edge_index[0], jnp.zeros((epad,), jnp.int32)]).reshape(ng, EGROUP)
    dst = jnp.concatenate(
        [edge_index[1], jnp.full((epad,), n, jnp.int32)]).reshape(ng, EGROUP)
    batch3 = batch.reshape(n // bn, 1, bn)
    b1r = b1.reshape(1, -1)
    b2r = b2.reshape(1, -1)
    bfcr = bfc.reshape(1, -1)

    deg_k = _make_deg_kernel(n, ng)
    agg_k = _make_agg_kernel(n, ng, h_dim)

    degp = deg_k(dst).reshape(NC, npad, LANE)
    h1p = _tc_matmul_scale(x, W1, degp, bn)
    acc1 = agg_k(src, dst, h1p).reshape(NC, npad, h_dim)
    h2p = _tc_mid(acc1, h1p, degp, b1r, W2, bn)
    acc2 = agg_k(src, dst, h2p).reshape(NC, npad, h_dim)
    return _tc_final(acc2, h2p, degp, b2r, batch3, Wfc, bfcr, bn)


# speedup vs baseline: 1.2351x; 1.2351x over previous
import functools

import jax
import jax.numpy as jnp
from jax import lax
from jax.experimental import pallas as pl
from jax.experimental.pallas import tpu as pltpu
from jax.experimental.pallas import tpu_sc as plsc

NC = 2
NS = 16
NW = NC * NS
LANE = 16
EGROUP = 128
NUM_GROUPS_POOL = 16


def _round_up(v, m):
    return -(-v // m) * m


def _sc_mesh():
    return plsc.VectorSubcoreMesh(core_axis_name="c", subcore_axis_name="s")


def _make_deg_kernel(n_nodes, n_groups):
    gpw = n_groups // NW
    npad = _round_up(n_nodes + 1, NS * 8)
    stripe = npad // NS

    @functools.partial(
        pl.kernel,
        out_type=jax.ShapeDtypeStruct((NC, NS, stripe, LANE), jnp.float32),
        mesh=_sc_mesh(),
        scratch_types=[
            pltpu.VMEM((gpw, EGROUP), jnp.int32),
            pltpu.VMEM((EGROUP, LANE), jnp.float32),
            pltpu.VMEM((EGROUP, LANE), jnp.float32),
            pltpu.VMEM_SHARED((npad, LANE), jnp.float32),
        ],
        compiler_params=pltpu.CompilerParams(use_tc_tiling_on_sc=False),
    )
    def deg_kernel(dst_hbm, out_hbm, dstbuf, ones_v, zeros_v, acc_sh):
        cid = lax.axis_index("c")
        sid = lax.axis_index("s")
        wid = sid * NC + cid

        def fill(i, carry):
            ones_v[i, :] = jnp.full((LANE,), 1.0, jnp.float32)
            zeros_v[i, :] = jnp.zeros((LANE,), jnp.float32)
            return carry
        lax.fori_loop(0, EGROUP, fill, 0)

        base = sid * stripe
        nfull = stripe // EGROUP
        for k in range(nfull):
            pltpu.sync_copy(zeros_v, acc_sh.at[pl.ds(base + k * EGROUP, EGROUP)])
        rem = stripe - nfull * EGROUP
        if rem:
            pltpu.sync_copy(zeros_v.at[pl.ds(0, rem)],
                            acc_sh.at[pl.ds(base + nfull * EGROUP, rem)])
        plsc.subcore_barrier()

        pltpu.sync_copy(dst_hbm.at[pl.ds(wid * gpw, gpw)], dstbuf)

        def step(j, carry):
            pltpu.sync_copy(ones_v, acc_sh.at[dstbuf.at[j]], add=True)
            return carry
        lax.fori_loop(0, gpw, step, 0)
        plsc.subcore_barrier()

        pltpu.sync_copy(acc_sh.at[pl.ds(base, stripe)],
                        out_hbm.at[cid].at[sid])

    return deg_kernel


def _make_agg_kernel(n_nodes, n_groups, h_dim):
    gpw = n_groups // NW
    npad = _round_up(n_nodes + 1, NS * 8)
    stripe = npad // NS

    @functools.partial(
        pl.kernel,
        out_type=jax.ShapeDtypeStruct((NC, NS, stripe, h_dim), jnp.float32),
        mesh=_sc_mesh(),
        scratch_types=[
            pltpu.VMEM((gpw, EGROUP), jnp.int32),
            pltpu.VMEM((gpw, EGROUP), jnp.int32),
            pltpu.VMEM((2, EGROUP, h_dim), jnp.float32),
            pltpu.VMEM_SHARED((npad, h_dim), jnp.float32),
            pltpu.VMEM_SHARED((npad, h_dim), jnp.float32),
            pltpu.SemaphoreType.DMA,
            pltpu.SemaphoreType.DMA,
        ],
        compiler_params=pltpu.CompilerParams(use_tc_tiling_on_sc=False),
    )
    def agg_kernel(src_hbm, dst_hbm, h_hbm, out_hbm,
                   srcbuf, dstbuf, rows_v, acc_sh, h_sh,
                   sem_a, sem_b):
        cid = lax.axis_index("c")
        sid = lax.axis_index("s")
        wid = sid * NC + cid

        def zfill(i, carry):
            for k in range(h_dim // LANE):
                rows_v[0, i, pl.ds(k * LANE, LANE)] = jnp.zeros((LANE,), jnp.float32)
            return carry
        lax.fori_loop(0, EGROUP, zfill, 0)

        base = sid * stripe
        nfull = stripe // EGROUP
        for k in range(nfull):
            pltpu.sync_copy(rows_v.at[0],
                            acc_sh.at[pl.ds(base + k * EGROUP, EGROUP)])
        rem = stripe - nfull * EGROUP
        if rem:
            pltpu.sync_copy(rows_v.at[0].at[pl.ds(0, rem)],
                            acc_sh.at[pl.ds(base + nfull * EGROUP, rem)])

        nfull_h = n_nodes // EGROUP
        tail_h = n_nodes - nfull_h * EGROUP
        nck = -(-(nfull_h + (1 if tail_h else 0)) // NS)
        for kk in range(nck):
            k = kk * NS + sid

            @pl.when(k < nfull_h)
            def _():
                pltpu.sync_copy(h_hbm.at[pl.ds(k * EGROUP, EGROUP)],
                                h_sh.at[pl.ds(k * EGROUP, EGROUP)])
            if tail_h:
                @pl.when(k == nfull_h)
                def _():
                    pltpu.sync_copy(h_hbm.at[pl.ds(nfull_h * EGROUP, tail_h)],
                                    h_sh.at[pl.ds(nfull_h * EGROUP, tail_h)])
        plsc.subcore_barrier()

        pltpu.sync_copy(src_hbm.at[pl.ds(wid * gpw, gpw)], srcbuf)
        pltpu.sync_copy(dst_hbm.at[pl.ds(wid * gpw, gpw)], dstbuf)

        pltpu.async_copy(h_sh.at[srcbuf.at[0]], rows_v.at[0], sem_a)

        def step2(jj, carry):
            j = jj * 2
            pltpu.async_copy(h_sh.at[srcbuf.at[j + 1]], rows_v.at[1], sem_b)
            pltpu.make_async_copy(h_sh.at[srcbuf.at[j]],
                                  rows_v.at[0], sem_a).wait()
            pltpu.sync_copy(rows_v.at[0], acc_sh.at[dstbuf.at[j]], add=True)

            @pl.when(jj + 1 < gpw // 2)
            def _():
                pltpu.async_copy(h_sh.at[srcbuf.at[j + 2]],
                                 rows_v.at[0], sem_a)
            pltpu.make_async_copy(h_sh.at[srcbuf.at[j + 1]],
                                  rows_v.at[1], sem_b).wait()
            pltpu.sync_copy(rows_v.at[1], acc_sh.at[dstbuf.at[j + 1]], add=True)
            return carry
        lax.fori_loop(0, gpw // 2, step2, 0)
        plsc.subcore_barrier()

        pltpu.sync_copy(acc_sh.at[pl.ds(base, stripe)],
                        out_hbm.at[cid].at[sid])

    return agg_kernel


def _dinv_from(deg_ref):
    deg = deg_ref[0, :, 0] + deg_ref[1, :, 0] + 1.0
    return lax.rsqrt(deg)


def _tc_matmul_scale(x, w, degp, bn):
    n, d = x.shape
    h = w.shape[1]

    def body(x_ref, w_ref, deg_ref, o_ref):
        dinv = _dinv_from(deg_ref)
        acc = jnp.dot(x_ref[...], w_ref[...], preferred_element_type=jnp.float32)
        o_ref[...] = acc * dinv[:, None]

    return pl.pallas_call(
        body,
        grid=(n // bn,),
        in_specs=[
            pl.BlockSpec((bn, d), lambda i: (i, 0)),
            pl.BlockSpec((d, h), lambda i: (0, 0)),
            pl.BlockSpec((2, bn, LANE), lambda i: (0, i, 0)),
        ],
        out_specs=pl.BlockSpec((bn, h), lambda i: (i, 0)),
        out_shape=jax.ShapeDtypeStruct((n, h), jnp.float32),
    )(x, w, degp)


def _tc_mid(accp, hp, degp, b, w2, bn):
    n, h = hp.shape
    h2 = w2.shape[1]

    def body(acc_ref, hp_ref, deg_ref, b_ref, w_ref, o_ref):
        dinv = _dinv_from(deg_ref)
        s = (acc_ref[0] + acc_ref[1] + hp_ref[...]) * dinv[:, None] + b_ref[...]
        hrelu = jnp.maximum(s, 0.0)
        o_ref[...] = jnp.dot(hrelu, w_ref[...],
                             preferred_element_type=jnp.float32) * dinv[:, None]

    return pl.pallas_call(
        body,
        grid=(n // bn,),
        in_specs=[
            pl.BlockSpec((2, bn, h), lambda i: (0, i, 0)),
            pl.BlockSpec((bn, h), lambda i: (i, 0)),
            pl.BlockSpec((2, bn, LANE), lambda i: (0, i, 0)),
            pl.BlockSpec((1, h), lambda i: (0, 0)),
            pl.BlockSpec((h, h2), lambda i: (0, 0)),
        ],
        out_specs=pl.BlockSpec((bn, h), lambda i: (i, 0)),
        out_shape=jax.ShapeDtypeStruct((n, h), jnp.float32),
    )(accp, hp, degp, b, w2)


def _tc_final(accp, hp, degp, b, batch3, wfc, bfc, bn):
    n, h = hp.shape
    c = wfc.shape[1]
    g = NUM_GROUPS_POOL
    nblk = n // bn

    def body(acc_ref, hp_ref, deg_ref, b_ref, batch_ref, wfc_ref, bfc_ref,
             o_ref, psum, cnt):
        i = pl.program_id(0)
        dinv = _dinv_from(deg_ref)
        s = (acc_ref[0] + acc_ref[1] + hp_ref[...]) * dinv[:, None] + b_ref[...]
        hrelu = jnp.maximum(s, 0.0)
        bvec = batch_ref[0, 0, :]
        oh = (bvec[:, None] == lax.broadcasted_iota(jnp.int32, (1, g), 1)
              ).astype(jnp.float32)
        contrib = lax.dot_general(oh, hrelu, (((0,), (0,)), ((), ())),
                                  preferred_element_type=jnp.float32)
        cnt_c = jnp.sum(oh, axis=0)[None, :]

        @pl.when(i == 0)
        def _():
            psum[...] = jnp.zeros_like(psum)
            cnt[...] = jnp.zeros_like(cnt)

        psum[...] += contrib
        cnt[...] += cnt_c

        @pl.when(i == nblk - 1)
        def _():
            pooled = psum[...] / jnp.maximum(cnt[...], 1.0)[0][:, None]
            o_ref[...] = jnp.dot(pooled, wfc_ref[...],
                                 preferred_element_type=jnp.float32) + bfc_ref[...]

    return pl.pallas_call(
        body,
        grid=(nblk,),
        in_specs=[
            pl.BlockSpec((2, bn, h), lambda i: (0, i, 0)),
            pl.BlockSpec((bn, h), lambda i: (i, 0)),
            pl.BlockSpec((2, bn, LANE), lambda i: (0, i, 0)),
            pl.BlockSpec((1, h), lambda i: (0, 0)),
            pl.BlockSpec((1, 1, bn), lambda i: (i, 0, 0)),
            pl.BlockSpec((h, c), lambda i: (0, 0)),
            pl.BlockSpec((1, c), lambda i: (0, 0)),
        ],
        out_specs=pl.BlockSpec((g, c), lambda i: (0, 0)),
        out_shape=jax.ShapeDtypeStruct((g, c), jnp.float32),
        scratch_shapes=[
            pltpu.VMEM((g, h), jnp.float32),
            pltpu.VMEM((1, g), jnp.float32),
        ],
    )(accp, hp, degp, b, batch3, wfc, bfc)


def kernel(x, edge_index, batch, W1, b1, W2, b2, Wfc, bfc):
    n, d = x.shape
    h_dim = W1.shape[1]
    e = edge_index.shape[1]
    bn = 2000

    ng = _round_up(_round_up(e, EGROUP) // EGROUP, NW * 8)
    epad = ng * EGROUP - e
    npad = _round_up(n + 1, NS * 8)
    src = jnp.concatenate(
        [edge_index[0], jnp.zeros((epad,), jnp.int32)]).reshape(ng, EGROUP)
    dst = jnp.concatenate(
        [edge_index[1], jnp.full((epad,), n, jnp.int32)]).reshape(ng, EGROUP)
    batch3 = batch.reshape(n // bn, 1, bn)
    b1r = b1.reshape(1, -1)
    b2r = b2.reshape(1, -1)
    bfcr = bfc.reshape(1, -1)

    deg_k = _make_deg_kernel(n, ng)
    agg_k = _make_agg_kernel(n, ng, h_dim)

    degp = deg_k(dst).reshape(NC, npad, LANE)
    h1p = _tc_matmul_scale(x, W1, degp, bn)
    acc1 = agg_k(src, dst, h1p).reshape(NC, npad, h_dim)
    h2p = _tc_mid(acc1, h1p, degp, b1r, W2, bn)
    acc2 = agg_k(src, dst, h2p).reshape(NC, npad, h_dim)
    return _tc_final(acc2, h2p, degp, b2r, batch3, Wfc, bfcr, bn)
